# baseline (speedup 1.0000x reference)
import functools

import jax
import jax.numpy as jnp
from jax import lax
from jax.experimental import pallas as pl
from jax.experimental.pallas import tpu as pltpu
from jax.experimental.pallas import tpu_sc as plsc

E = 8
K = 2
T = 2048
DIN = 1024
DFF = 2048
DOUT = 1024
A = K * T
NW = 32
CHUNK = A // NW
BM = 512
NT = A // BM
NSTEPS = NT + E
APAD = NSTEPS * BM


def _gating_body(x_ref, wg_ref, bg_ref, dest_ref, w16_ref, off_ref):
    x = x_ref[...]
    wg = wg_ref[...]
    logits = jnp.dot(x, wg, preferred_element_type=jnp.float32) + bg_ref[...]
    m = jnp.max(logits, axis=-1, keepdims=True)
    p = jnp.exp(logits - m)
    p = p / jnp.sum(p, axis=-1, keepdims=True)

    iota_e = lax.broadcasted_iota(jnp.int32, (T, E), 1)
    m1 = jnp.max(p, axis=-1, keepdims=True)
    i1 = jnp.min(jnp.where(p == m1, iota_e, E), axis=-1)
    p2 = jnp.where(iota_e == i1[:, None], -1.0, p)
    m2 = jnp.max(p2, axis=-1, keepdims=True)
    i2 = jnp.min(jnp.where(p2 == m2, iota_e, E), axis=-1)

    w16_ref[0, :, :] = jnp.broadcast_to(m1, (T, 128))
    w16_ref[1, :, :] = jnp.broadcast_to(m2, (T, 128))

    oh1 = (i1[:, None] == iota_e).astype(jnp.float32)
    oh2 = (i2[:, None] == iota_e).astype(jnp.float32)
    oh = jnp.concatenate([oh1, oh2], axis=0)

    c = oh
    k = 1
    while k < A:
        shifted = jnp.concatenate(
            [jnp.zeros((k, E), jnp.float32), c[:A - k]], axis=0)
        c = c + shifted
        k *= 2
    rank = jnp.sum(c * oh, axis=-1) - 1.0

    tot = jnp.sum(oh, axis=0, keepdims=True)
    totp = jnp.ceil(tot * (1.0 / BM)) * BM
    inc = totp
    k = 1
    while k < E:
        inc = inc + jnp.concatenate(
            [jnp.zeros((1, k), jnp.float32), inc[:, :E - k]], axis=1)
        k *= 2
    goff = inc - totp

    dest = rank + jnp.sum(goff * oh, axis=-1)
    dest_ref[...] = dest.astype(jnp.int32).reshape(K, T)

    lane = lax.broadcasted_iota(jnp.int32, (1, 16), 1)
    endp = (goff + totp).astype(jnp.int32)[:, E - 1:E]
    offs9 = jnp.concatenate(
        [goff.astype(jnp.int32), endp, jnp.full((1, 7), 0, jnp.int32)], axis=1)
    off_ref[...] = jnp.where(lane < 9, offs9, 0)


def _gating(x, Wg, bg):
    return pl.pallas_call(
        _gating_body,
        out_shape=[
            jax.ShapeDtypeStruct((K, T), jnp.int32),
            jax.ShapeDtypeStruct((K, T, 128), jnp.float32),
            jax.ShapeDtypeStruct((1, 16), jnp.int32),
        ],
    )(x, Wg, bg.reshape(1, E))


def _dispatch_body(dest3_hbm, dest2_hbm, x_hbm, w16_hbm, xs_hbm, ws_hbm,
                   dest2d_v, dest128_v, rows0_v, rows1_v, wrow_v,
                   lsem, ssem, wsem):
    wid = lax.axis_index("s") * 2 + lax.axis_index("c")
    base = wid * CHUNK
    tok_base = lax.rem(base, T)
    bufs = (rows0_v, rows1_v)

    pltpu.sync_copy(dest3_hbm.at[wid], dest2d_v)
    pltpu.sync_copy(dest2_hbm.at[wid], dest128_v)

    wload = pltpu.async_copy(w16_hbm.at[pl.ds(base, CHUNK)], wrow_v, wsem)

    loads = [pltpu.async_copy(x_hbm.at[pl.ds(tok_base, 32)], bufs[0], lsem)]
    scats = []
    for c in range(4):
        loads[c].wait()
        if c >= 1:
            scats[c - 1].wait()
        if c < 3:
            loads.append(pltpu.async_copy(
                x_hbm.at[pl.ds(tok_base + (c + 1) * 32, 32)],
                bufs[(c + 1) % 2], lsem))
        scats.append(pltpu.async_copy(
            bufs[c % 2], xs_hbm.at[dest2d_v.at[c]], ssem))
    wload.wait()
    wscat = pltpu.async_copy(wrow_v, ws_hbm.at[dest128_v], wsem)
    scats[3].wait()
    wscat.wait()


def _dispatch(dest3, dest2, x, w16_flat):
    mesh = plsc.VectorSubcoreMesh(core_axis_name="c", subcore_axis_name="s")
    f = functools.partial(
        pl.kernel,
        out_type=[
            jax.ShapeDtypeStruct((APAD, DIN), jnp.float32),
            jax.ShapeDtypeStruct((APAD, 128), jnp.float32),
        ],
        mesh=mesh,
        scratch_types=[
            pltpu.VMEM((4, 32), jnp.int32),
            pltpu.VMEM((CHUNK,), jnp.int32),
            pltpu.VMEM((32, DIN), jnp.float32),
            pltpu.VMEM((32, DIN), jnp.float32),
            pltpu.VMEM((CHUNK, 128), jnp.float32),
            pltpu.SemaphoreType.DMA,
            pltpu.SemaphoreType.DMA,
            pltpu.SemaphoreType.DMA,
        ],
    )(_dispatch_body)
    return f(dest3, dest2, x, w16_flat)


def _gmm_body(gid_ref, tid_ref, valid_ref,
              ws_ref, xs_ref, w1_ref, b1_ref, w2_ref, b2_ref, ys_ref):
    s = pl.program_id(0)

    @pl.when(valid_ref[s] == 1)
    def _go():
        xb = xs_ref[...].astype(jnp.bfloat16)
        h = jnp.dot(xb, w1_ref[0], preferred_element_type=jnp.float32)
        h = jnp.maximum(h + b1_ref[0], 0.0).astype(jnp.bfloat16)
        y = jnp.dot(h, w2_ref[0], preferred_element_type=jnp.float32)
        y = y + b2_ref[0]
        ys_ref[...] = y * ws_ref[:, :1]


def _gmm(gid, tid, valid, ws, xs, W1, b1, W2, b2):
    grid_spec = pltpu.PrefetchScalarGridSpec(
        num_scalar_prefetch=3,
        grid=(NSTEPS,),
        in_specs=[
            pl.BlockSpec((BM, 128), lambda s, g, t, v: (t[s], 0)),
            pl.BlockSpec((BM, DIN), lambda s, g, t, v: (t[s], 0)),
            pl.BlockSpec((1, DIN, DFF), lambda s, g, t, v: (g[s], 0, 0)),
            pl.BlockSpec((1, 1, DFF), lambda s, g, t, v: (g[s], 0, 0)),
            pl.BlockSpec((1, DFF, DOUT), lambda s, g, t, v: (g[s], 0, 0)),
            pl.BlockSpec((1, 1, DOUT), lambda s, g, t, v: (g[s], 0, 0)),
        ],
        out_specs=pl.BlockSpec((BM, DOUT), lambda s, g, t, v: (t[s], 0)),
    )
    return pl.pallas_call(
        _gmm_body,
        grid_spec=grid_spec,
        out_shape=jax.ShapeDtypeStruct((APAD, DOUT), jnp.float32),
        compiler_params=pltpu.CompilerParams(
            dimension_semantics=("arbitrary",)),
    )(gid, tid, valid, ws, xs,
      W1.astype(jnp.bfloat16), b1.reshape(E, 1, DFF),
      W2.astype(jnp.bfloat16), b2.reshape(E, 1, DOUT))


def _schedule(off9):
    cum = off9[1:E + 1] // BM
    total = cum[E - 1]
    s = jnp.arange(NSTEPS, dtype=jnp.int32)
    e_s = jnp.searchsorted(cum, s, side="right").astype(jnp.int32)
    valid = (s < total).astype(jnp.int32)
    gid_last = jnp.max(jnp.where(valid == 1, jnp.minimum(e_s, E - 1), 0))
    gid = jnp.where(valid == 1, jnp.minimum(e_s, E - 1), gid_last)
    tid = jnp.minimum(s, jnp.maximum(total - 1, 0)).astype(jnp.int32)
    return gid.astype(jnp.int32), tid, valid


def _combine_body(ys_hbm, inv1_hbm, inv2_hbm, out_hbm,
                  inv1_v, inv2_v, y1a_v, y2a_v, y1b_v, y2b_v, gsem, osem):
    wid = lax.axis_index("s") * 2 + lax.axis_index("c")
    tb = wid * (T // NW)
    CH = 16
    y1 = (y1a_v, y1b_v)
    y2 = (y2a_v, y2b_v)

    pltpu.sync_copy(inv1_hbm.at[wid], inv1_v)
    pltpu.sync_copy(inv2_hbm.at[wid], inv2_v)

    gath = [pltpu.async_copy(ys_hbm.at[inv1_v.at[0]], y1[0], gsem),
            pltpu.async_copy(ys_hbm.at[inv2_v.at[0]], y2[0], gsem)]
    stores = []
    for c in range(4):
        gath[2 * c].wait()
        gath[2 * c + 1].wait()
        if c >= 1:
            stores[c - 1].wait()
        if c < 3:
            gath.append(pltpu.async_copy(
                ys_hbm.at[inv1_v.at[c + 1]], y1[(c + 1) % 2], gsem))
            gath.append(pltpu.async_copy(
                ys_hbm.at[inv2_v.at[c + 1]], y2[(c + 1) % 2], gsem))
        y1c, y2c = y1[c % 2], y2[c % 2]

        def body(r, _):
            for sseg in range(DOUT // 16):
                sl = pl.ds(sseg * 16, 16)
                y1c[r, sl] = y1c[r, sl] + y2c[r, sl]
            return 0

        lax.fori_loop(0, CH, body, 0)
        stores.append(pltpu.async_copy(
            y1c, out_hbm.at[pl.ds(tb + c * CH, CH)], osem))
    stores[3].wait()


def _combine(ys, inv1, inv2):
    mesh = plsc.VectorSubcoreMesh(core_axis_name="c", subcore_axis_name="s")
    f = functools.partial(
        pl.kernel,
        out_type=jax.ShapeDtypeStruct((T, DOUT), jnp.float32),
        mesh=mesh,
        scratch_types=[
            pltpu.VMEM((4, 16), jnp.int32),
            pltpu.VMEM((4, 16), jnp.int32),
            pltpu.VMEM((16, DOUT), jnp.float32),
            pltpu.VMEM((16, DOUT), jnp.float32),
            pltpu.VMEM((16, DOUT), jnp.float32),
            pltpu.VMEM((16, DOUT), jnp.float32),
            pltpu.SemaphoreType.DMA,
            pltpu.SemaphoreType.DMA,
        ],
    )(_combine_body)
    return f(ys, inv1, inv2)


def kernel(x, Wg, bg, W1, b1, W2, b2):
    dest_km, w16, off16 = _gating(x, Wg, bg)
    dest_flat = dest_km.reshape(A)
    dest3 = dest_km.reshape(NW, 4, 32)
    dest2 = dest_km.reshape(NW, CHUNK)
    w16_flat = w16.reshape(A, 128)
    xs, ws = _dispatch(dest3, dest2, x, w16_flat)
    off9 = off16.reshape(16)[:E + 1]
    gid, tid, valid = _schedule(off9)
    ys = _gmm(gid, tid, valid, ws, xs, W1, b1, W2, b2)
    inv1 = dest_flat[:T].reshape(NW, 4, 16)
    inv2 = dest_flat[T:].reshape(NW, 4, 16)
    return _combine(ys, inv1, inv2)


# speedup vs baseline: 1.3275x; 1.3275x over previous
import functools

import jax
import jax.numpy as jnp
from jax import lax
from jax.experimental import pallas as pl
from jax.experimental.pallas import tpu as pltpu
from jax.experimental.pallas import tpu_sc as plsc

E = 8
K = 2
T = 2048
DIN = 1024
DFF = 2048
DOUT = 1024
A = K * T
NW = 32
CHUNK = A // NW
BM = 1024
NT = A // BM
NSTEPS = NT + E
APAD = NSTEPS * BM


def _gating_body(x_ref, wg_ref, bg_ref, dest_ref, w16_ref, off_ref):
    x = x_ref[...]
    wg = wg_ref[...]
    logits = jnp.dot(x, wg, preferred_element_type=jnp.float32) + bg_ref[...]
    m = jnp.max(logits, axis=-1, keepdims=True)
    p = jnp.exp(logits - m)
    p = p / jnp.sum(p, axis=-1, keepdims=True)

    iota_e = lax.broadcasted_iota(jnp.int32, (T, E), 1)
    m1 = jnp.max(p, axis=-1, keepdims=True)
    i1 = jnp.min(jnp.where(p == m1, iota_e, E), axis=-1)
    p2 = jnp.where(iota_e == i1[:, None], -1.0, p)
    m2 = jnp.max(p2, axis=-1, keepdims=True)
    i2 = jnp.min(jnp.where(p2 == m2, iota_e, E), axis=-1)

    w16_ref[0, :, :] = jnp.broadcast_to(m1, (T, 128))
    w16_ref[1, :, :] = jnp.broadcast_to(m2, (T, 128))

    oh1 = (i1[:, None] == iota_e).astype(jnp.float32)
    oh2 = (i2[:, None] == iota_e).astype(jnp.float32)
    oh = jnp.concatenate([oh1, oh2], axis=0)

    c = oh
    k = 1
    while k < A:
        shifted = jnp.concatenate(
            [jnp.zeros((k, E), jnp.float32), c[:A - k]], axis=0)
        c = c + shifted
        k *= 2
    rank = jnp.sum(c * oh, axis=-1) - 1.0

    tot = jnp.sum(oh, axis=0, keepdims=True)
    totp = jnp.ceil(tot * (1.0 / BM)) * BM
    inc = totp
    k = 1
    while k < E:
        inc = inc + jnp.concatenate(
            [jnp.zeros((1, k), jnp.float32), inc[:, :E - k]], axis=1)
        k *= 2
    goff = inc - totp

    dest = rank + jnp.sum(goff * oh, axis=-1)
    dest_ref[...] = dest.astype(jnp.int32).reshape(K, T)

    lane = lax.broadcasted_iota(jnp.int32, (1, 16), 1)
    endp = (goff + totp).astype(jnp.int32)[:, E - 1:E]
    offs9 = jnp.concatenate(
        [goff.astype(jnp.int32), endp, jnp.full((1, 7), 0, jnp.int32)], axis=1)
    off_ref[...] = jnp.where(lane < 9, offs9, 0)


def _gating(x, Wg, bg):
    return pl.pallas_call(
        _gating_body,
        out_shape=[
            jax.ShapeDtypeStruct((K, T), jnp.int32),
            jax.ShapeDtypeStruct((K, T, 128), jnp.float32),
            jax.ShapeDtypeStruct((1, 16), jnp.int32),
        ],
    )(x, Wg, bg.reshape(1, E))


def _dispatch_body(dest3_hbm, dest2_hbm, x_hbm, w16_hbm, xs_hbm, ws_hbm,
                   dest2d_v, dest128_v, rows0_v, rows1_v, wrow_v,
                   lsem, ssem, wsem):
    wid = lax.axis_index("s") * 2 + lax.axis_index("c")
    base = wid * CHUNK
    tok_base = lax.rem(base, T)
    bufs = (rows0_v, rows1_v)

    pltpu.sync_copy(dest3_hbm.at[wid], dest2d_v)
    pltpu.sync_copy(dest2_hbm.at[wid], dest128_v)

    wload = pltpu.async_copy(w16_hbm.at[pl.ds(base, CHUNK)], wrow_v, wsem)

    loads = [pltpu.async_copy(x_hbm.at[pl.ds(tok_base, 32)], bufs[0], lsem)]
    scats = []
    for c in range(4):
        loads[c].wait()
        if c >= 1:
            scats[c - 1].wait()
        if c < 3:
            loads.append(pltpu.async_copy(
                x_hbm.at[pl.ds(tok_base + (c + 1) * 32, 32)],
                bufs[(c + 1) % 2], lsem))
        scats.append(pltpu.async_copy(
            bufs[c % 2], xs_hbm.at[dest2d_v.at[c]], ssem))
    wload.wait()
    wscat = pltpu.async_copy(wrow_v, ws_hbm.at[dest128_v], wsem)
    scats[3].wait()
    wscat.wait()


def _dispatch(dest3, dest2, x, w16_flat):
    mesh = plsc.VectorSubcoreMesh(core_axis_name="c", subcore_axis_name="s")
    f = functools.partial(
        pl.kernel,
        out_type=[
            jax.ShapeDtypeStruct((APAD, DIN), jnp.float32),
            jax.ShapeDtypeStruct((APAD, 128), jnp.float32),
        ],
        mesh=mesh,
        scratch_types=[
            pltpu.VMEM((4, 32), jnp.int32),
            pltpu.VMEM((CHUNK,), jnp.int32),
            pltpu.VMEM((32, DIN), jnp.float32),
            pltpu.VMEM((32, DIN), jnp.float32),
            pltpu.VMEM((CHUNK, 128), jnp.float32),
            pltpu.SemaphoreType.DMA,
            pltpu.SemaphoreType.DMA,
            pltpu.SemaphoreType.DMA,
        ],
    )(_dispatch_body)
    return f(dest3, dest2, x, w16_flat)


def _gmm_body(gid_ref, tid_ref, valid_ref,
              ws_ref, xs_ref, w1_ref, b1_ref, w2_ref, b2_ref, ys_ref):
    s = pl.program_id(0)

    @pl.when(valid_ref[s] == 1)
    def _go():
        xb = xs_ref[...]
        h = jnp.dot(xb, w1_ref[0], preferred_element_type=jnp.float32)
        h = jnp.maximum(h + b1_ref[0], 0.0)
        y = jnp.dot(h, w2_ref[0], preferred_element_type=jnp.float32)
        y = y + b2_ref[0]
        ys_ref[...] = y * ws_ref[:, :1]


def _gmm(gid, tid, valid, ws, xs, W1, b1, W2, b2):
    grid_spec = pltpu.PrefetchScalarGridSpec(
        num_scalar_prefetch=3,
        grid=(NSTEPS,),
        in_specs=[
            pl.BlockSpec((BM, 128), lambda s, g, t, v: (t[s], 0)),
            pl.BlockSpec((BM, DIN), lambda s, g, t, v: (t[s], 0)),
            pl.BlockSpec((1, DIN, DFF), lambda s, g, t, v: (g[s], 0, 0)),
            pl.BlockSpec((1, 1, DFF), lambda s, g, t, v: (g[s], 0, 0)),
            pl.BlockSpec((1, DFF, DOUT), lambda s, g, t, v: (g[s], 0, 0)),
            pl.BlockSpec((1, 1, DOUT), lambda s, g, t, v: (g[s], 0, 0)),
        ],
        out_specs=pl.BlockSpec((BM, DOUT), lambda s, g, t, v: (t[s], 0)),
    )
    return pl.pallas_call(
        _gmm_body,
        grid_spec=grid_spec,
        out_shape=jax.ShapeDtypeStruct((APAD, DOUT), jnp.float32),
        compiler_params=pltpu.CompilerParams(
            dimension_semantics=("arbitrary",)),
    )(gid, tid, valid, ws, xs,
      W1, b1.reshape(E, 1, DFF), W2, b2.reshape(E, 1, DOUT))


def _schedule(off9):
    cum = off9[1:E + 1] // BM
    total = cum[E - 1]
    s = jnp.arange(NSTEPS, dtype=jnp.int32)
    e_s = jnp.searchsorted(cum, s, side="right").astype(jnp.int32)
    valid = (s < total).astype(jnp.int32)
    gid_last = jnp.max(jnp.where(valid == 1, jnp.minimum(e_s, E - 1), 0))
    gid = jnp.where(valid == 1, jnp.minimum(e_s, E - 1), gid_last)
    tid = jnp.minimum(s, jnp.maximum(total - 1, 0)).astype(jnp.int32)
    return gid.astype(jnp.int32), tid, valid


def _combine_body(ys_hbm, inv1_hbm, inv2_hbm, out_hbm,
                  inv1_v, inv2_v, y1a_v, y2a_v, y1b_v, y2b_v, gsem, osem):
    wid = lax.axis_index("s") * 2 + lax.axis_index("c")
    tb = wid * (T // NW)
    CH = 16
    y1 = (y1a_v, y1b_v)
    y2 = (y2a_v, y2b_v)

    pltpu.sync_copy(inv1_hbm.at[wid], inv1_v)
    pltpu.sync_copy(inv2_hbm.at[wid], inv2_v)

    gath = [pltpu.async_copy(ys_hbm.at[inv1_v.at[0]], y1[0], gsem),
            pltpu.async_copy(ys_hbm.at[inv2_v.at[0]], y2[0], gsem)]
    stores = []
    for c in range(4):
        gath[2 * c].wait()
        gath[2 * c + 1].wait()
        if c >= 1:
            stores[c - 1].wait()
        if c < 3:
            gath.append(pltpu.async_copy(
                ys_hbm.at[inv1_v.at[c + 1]], y1[(c + 1) % 2], gsem))
            gath.append(pltpu.async_copy(
                ys_hbm.at[inv2_v.at[c + 1]], y2[(c + 1) % 2], gsem))
        y1c, y2c = y1[c % 2], y2[c % 2]

        def body(r, _):
            for sseg in range(DOUT // 16):
                sl = pl.ds(sseg * 16, 16)
                y1c[r, sl] = y1c[r, sl] + y2c[r, sl]
            return 0

        lax.fori_loop(0, CH, body, 0)
        stores.append(pltpu.async_copy(
            y1c, out_hbm.at[pl.ds(tb + c * CH, CH)], osem))
    stores[3].wait()


def _combine(ys, inv1, inv2):
    mesh = plsc.VectorSubcoreMesh(core_axis_name="c", subcore_axis_name="s")
    f = functools.partial(
        pl.kernel,
        out_type=jax.ShapeDtypeStruct((T, DOUT), jnp.float32),
        mesh=mesh,
        scratch_types=[
            pltpu.VMEM((4, 16), jnp.int32),
            pltpu.VMEM((4, 16), jnp.int32),
            pltpu.VMEM((16, DOUT), jnp.float32),
            pltpu.VMEM((16, DOUT), jnp.float32),
            pltpu.VMEM((16, DOUT), jnp.float32),
            pltpu.VMEM((16, DOUT), jnp.float32),
            pltpu.SemaphoreType.DMA,
            pltpu.SemaphoreType.DMA,
        ],
    )(_combine_body)
    return f(ys, inv1, inv2)


def kernel(x, Wg, bg, W1, b1, W2, b2):
    dest_km, w16, off16 = _gating(x, Wg, bg)
    dest_flat = dest_km.reshape(A)
    dest3 = dest_km.reshape(NW, 4, 32)
    dest2 = dest_km.reshape(NW, CHUNK)
    w16_flat = w16.reshape(A, 128)
    xs, ws = _dispatch(dest3, dest2, x, w16_flat)
    off9 = off16.reshape(16)[:E + 1]
    gid, tid, valid = _schedule(off9)
    ys = _gmm(gid, tid, valid, ws, xs, W1, b1, W2, b2)
    inv1 = dest_flat[:T].reshape(NW, 4, 16)
    inv2 = dest_flat[T:].reshape(NW, 4, 16)
    return _combine(ys, inv1, inv2)


# speedup vs baseline: 1.3672x; 1.0299x over previous
import functools

import jax
import jax.numpy as jnp
from jax import lax
from jax.experimental import pallas as pl
from jax.experimental.pallas import tpu as pltpu
from jax.experimental.pallas import tpu_sc as plsc

E = 8
K = 2
T = 2048
DIN = 1024
DFF = 2048
DOUT = 1024
A = K * T
NW = 32
CHUNK = A // NW
BM = 512
NT = A // BM
NSTEPS = NT + E
APAD = NSTEPS * BM


def _gating_body(x_ref, wg_ref, bg_ref, dest_ref, w16_ref, off_ref):
    x = x_ref[...]
    wg = wg_ref[...]
    logits = jnp.dot(x, wg, preferred_element_type=jnp.float32) + bg_ref[...]
    m = jnp.max(logits, axis=-1, keepdims=True)
    p = jnp.exp(logits - m)
    p = p / jnp.sum(p, axis=-1, keepdims=True)

    iota_e = lax.broadcasted_iota(jnp.int32, (T, E), 1)
    m1 = jnp.max(p, axis=-1, keepdims=True)
    i1 = jnp.min(jnp.where(p == m1, iota_e, E), axis=-1)
    p2 = jnp.where(iota_e == i1[:, None], -1.0, p)
    m2 = jnp.max(p2, axis=-1, keepdims=True)
    i2 = jnp.min(jnp.where(p2 == m2, iota_e, E), axis=-1)

    w16_ref[0, :, :] = jnp.broadcast_to(m1, (T, 128))
    w16_ref[1, :, :] = jnp.broadcast_to(m2, (T, 128))

    oh1 = (i1[:, None] == iota_e).astype(jnp.float32)
    oh2 = (i2[:, None] == iota_e).astype(jnp.float32)
    oh = jnp.concatenate([oh1, oh2], axis=0)

    c = oh
    k = 1
    while k < A:
        shifted = jnp.concatenate(
            [jnp.zeros((k, E), jnp.float32), c[:A - k]], axis=0)
        c = c + shifted
        k *= 2
    rank = jnp.sum(c * oh, axis=-1) - 1.0

    tot = jnp.sum(oh, axis=0, keepdims=True)
    totp = jnp.ceil(tot * (1.0 / BM)) * BM
    inc = totp
    k = 1
    while k < E:
        inc = inc + jnp.concatenate(
            [jnp.zeros((1, k), jnp.float32), inc[:, :E - k]], axis=1)
        k *= 2
    goff = inc - totp

    dest = rank + jnp.sum(goff * oh, axis=-1)
    dest_ref[...] = dest.astype(jnp.int32).reshape(K, T)

    lane = lax.broadcasted_iota(jnp.int32, (1, 16), 1)
    endp = (goff + totp).astype(jnp.int32)[:, E - 1:E]
    offs9 = jnp.concatenate(
        [goff.astype(jnp.int32), endp, jnp.full((1, 7), 0, jnp.int32)], axis=1)
    off_ref[...] = jnp.where(lane < 9, offs9, 0)


def _gating(x, Wg, bg):
    return pl.pallas_call(
        _gating_body,
        out_shape=[
            jax.ShapeDtypeStruct((K, T), jnp.int32),
            jax.ShapeDtypeStruct((K, T, 128), jnp.float32),
            jax.ShapeDtypeStruct((1, 16), jnp.int32),
        ],
    )(x, Wg, bg.reshape(1, E))


def _dispatch_body(dest3_hbm, dest2_hbm, x_hbm, w16_hbm, xs_hbm, ws_hbm,
                   dest2d_v, dest128_v, rows0_v, rows1_v, wrow_v,
                   lsem, ssem, wsem):
    wid = lax.axis_index("s") * 2 + lax.axis_index("c")
    base = wid * CHUNK
    tok_base = lax.rem(base, T)
    bufs = (rows0_v, rows1_v)

    pltpu.sync_copy(dest3_hbm.at[wid], dest2d_v)
    pltpu.sync_copy(dest2_hbm.at[wid], dest128_v)

    wload = pltpu.async_copy(w16_hbm.at[pl.ds(base, CHUNK)], wrow_v, wsem)

    loads = [pltpu.async_copy(x_hbm.at[pl.ds(tok_base, 32)], bufs[0], lsem)]
    scats = []
    for c in range(4):
        loads[c].wait()
        if c >= 1:
            scats[c - 1].wait()
        if c < 3:
            loads.append(pltpu.async_copy(
                x_hbm.at[pl.ds(tok_base + (c + 1) * 32, 32)],
                bufs[(c + 1) % 2], lsem))
        scats.append(pltpu.async_copy(
            bufs[c % 2], xs_hbm.at[dest2d_v.at[c]], ssem))
    wload.wait()
    wscat = pltpu.async_copy(wrow_v, ws_hbm.at[dest128_v], wsem)
    scats[3].wait()
    wscat.wait()


def _dispatch(dest3, dest2, x, w16_flat):
    mesh = plsc.VectorSubcoreMesh(core_axis_name="c", subcore_axis_name="s")
    f = functools.partial(
        pl.kernel,
        out_type=[
            jax.ShapeDtypeStruct((APAD, DIN), jnp.float32),
            jax.ShapeDtypeStruct((APAD, 128), jnp.float32),
        ],
        mesh=mesh,
        scratch_types=[
            pltpu.VMEM((4, 32), jnp.int32),
            pltpu.VMEM((CHUNK,), jnp.int32),
            pltpu.VMEM((32, DIN), jnp.float32),
            pltpu.VMEM((32, DIN), jnp.float32),
            pltpu.VMEM((CHUNK, 128), jnp.float32),
            pltpu.SemaphoreType.DMA,
            pltpu.SemaphoreType.DMA,
            pltpu.SemaphoreType.DMA,
        ],
    )(_dispatch_body)
    return f(dest3, dest2, x, w16_flat)


def _gmm_body(gid_ref, tid_ref, valid_ref,
              ws_ref, xs_ref, w1_ref, b1_ref, w2_ref, b2_ref, ys_ref):
    s = pl.program_id(0)

    @pl.when(valid_ref[s] == 1)
    def _go():
        xb = xs_ref[...]
        h = jnp.dot(xb, w1_ref[0], preferred_element_type=jnp.float32)
        h = jnp.maximum(h + b1_ref[0], 0.0)
        y = jnp.dot(h, w2_ref[0], preferred_element_type=jnp.float32)
        y = y + b2_ref[0]
        ys_ref[...] = y * ws_ref[:, :1]


def _gmm(gid, tid, valid, ws, xs, W1, b1, W2, b2):
    grid_spec = pltpu.PrefetchScalarGridSpec(
        num_scalar_prefetch=3,
        grid=(NSTEPS,),
        in_specs=[
            pl.BlockSpec((BM, 128), lambda s, g, t, v: (t[s], 0)),
            pl.BlockSpec((BM, DIN), lambda s, g, t, v: (t[s], 0)),
            pl.BlockSpec((1, DIN, DFF), lambda s, g, t, v: (g[s], 0, 0)),
            pl.BlockSpec((1, 1, DFF), lambda s, g, t, v: (g[s], 0, 0)),
            pl.BlockSpec((1, DFF, DOUT), lambda s, g, t, v: (g[s], 0, 0)),
            pl.BlockSpec((1, 1, DOUT), lambda s, g, t, v: (g[s], 0, 0)),
        ],
        out_specs=pl.BlockSpec((BM, DOUT), lambda s, g, t, v: (t[s], 0)),
    )
    return pl.pallas_call(
        _gmm_body,
        grid_spec=grid_spec,
        out_shape=jax.ShapeDtypeStruct((APAD, DOUT), jnp.float32),
        compiler_params=pltpu.CompilerParams(
            dimension_semantics=("arbitrary",)),
    )(gid, tid, valid, ws, xs,
      W1, b1.reshape(E, 1, DFF), W2, b2.reshape(E, 1, DOUT))


def _schedule(off9):
    cum = off9[1:E + 1] // BM
    total = cum[E - 1]
    s = jnp.arange(NSTEPS, dtype=jnp.int32)
    e_s = jnp.searchsorted(cum, s, side="right").astype(jnp.int32)
    valid = (s < total).astype(jnp.int32)
    gid_last = jnp.max(jnp.where(valid == 1, jnp.minimum(e_s, E - 1), 0))
    gid = jnp.where(valid == 1, jnp.minimum(e_s, E - 1), gid_last)
    tid = jnp.minimum(s, jnp.maximum(total - 1, 0)).astype(jnp.int32)
    return gid.astype(jnp.int32), tid, valid


def _combine_body(ys_hbm, inv1_hbm, inv2_hbm, out_hbm,
                  inv1_v, inv2_v, y1a_v, y2a_v, y1b_v, y2b_v, gsem, osem):
    wid = lax.axis_index("s") * 2 + lax.axis_index("c")
    tb = wid * (T // NW)
    CH = 16
    y1 = (y1a_v, y1b_v)
    y2 = (y2a_v, y2b_v)

    pltpu.sync_copy(inv1_hbm.at[wid], inv1_v)
    pltpu.sync_copy(inv2_hbm.at[wid], inv2_v)

    gath = [pltpu.async_copy(ys_hbm.at[inv1_v.at[0]], y1[0], gsem),
            pltpu.async_copy(ys_hbm.at[inv2_v.at[0]], y2[0], gsem)]
    stores = []
    for c in range(4):
        gath[2 * c].wait()
        gath[2 * c + 1].wait()
        if c >= 1:
            stores[c - 1].wait()
        if c < 3:
            gath.append(pltpu.async_copy(
                ys_hbm.at[inv1_v.at[c + 1]], y1[(c + 1) % 2], gsem))
            gath.append(pltpu.async_copy(
                ys_hbm.at[inv2_v.at[c + 1]], y2[(c + 1) % 2], gsem))
        y1c, y2c = y1[c % 2], y2[c % 2]

        def body(r, _):
            for sseg in range(DOUT // 16):
                sl = pl.ds(sseg * 16, 16)
                y1c[r, sl] = y1c[r, sl] + y2c[r, sl]
            return 0

        lax.fori_loop(0, CH, body, 0)
        stores.append(pltpu.async_copy(
            y1c, out_hbm.at[pl.ds(tb + c * CH, CH)], osem))
    stores[3].wait()


def _combine(ys, inv1, inv2):
    mesh = plsc.VectorSubcoreMesh(core_axis_name="c", subcore_axis_name="s")
    f = functools.partial(
        pl.kernel,
        out_type=jax.ShapeDtypeStruct((T, DOUT), jnp.float32),
        mesh=mesh,
        scratch_types=[
            pltpu.VMEM((4, 16), jnp.int32),
            pltpu.VMEM((4, 16), jnp.int32),
            pltpu.VMEM((16, DOUT), jnp.float32),
            pltpu.VMEM((16, DOUT), jnp.float32),
            pltpu.VMEM((16, DOUT), jnp.float32),
            pltpu.VMEM((16, DOUT), jnp.float32),
            pltpu.SemaphoreType.DMA,
            pltpu.SemaphoreType.DMA,
        ],
    )(_combine_body)
    return f(ys, inv1, inv2)


def kernel(x, Wg, bg, W1, b1, W2, b2):
    dest_km, w16, off16 = _gating(x, Wg, bg)
    dest_flat = dest_km.reshape(A)
    dest3 = dest_km.reshape(NW, 4, 32)
    dest2 = dest_km.reshape(NW, CHUNK)
    w16_flat = w16.reshape(A, 128)
    xs, ws = _dispatch(dest3, dest2, x, w16_flat)
    off9 = off16.reshape(16)[:E + 1]
    gid, tid, valid = _schedule(off9)
    ys = _gmm(gid, tid, valid, ws, xs, W1, b1, W2, b2)
    inv1 = dest_flat[:T].reshape(NW, 4, 16)
    inv2 = dest_flat[T:].reshape(NW, 4, 16)
    return _combine(ys, inv1, inv2)
